# baseline (speedup 1.0000x reference)
import jax
import jax.numpy as jnp
from jax.experimental import pallas as pl

_N_TOK = 16384
_D = 2048
_E = 16
_BT = 2048
_NS = 4
_DC = _D // _NS


def _gating_body(x0_ref, x1_ref, x2_ref, x3_ref, wt_ref, nb_ref, o_ref):
    f32 = jnp.float32
    logits = (jnp.dot(x0_ref[...], wt_ref[pl.ds(0 * _DC, _DC), :],
                      preferred_element_type=f32)
              + jnp.dot(x1_ref[...], wt_ref[pl.ds(1 * _DC, _DC), :],
                        preferred_element_type=f32)
              + jnp.dot(x2_ref[...], wt_ref[pl.ds(2 * _DC, _DC), :],
                        preferred_element_type=f32)
              + jnp.dot(x3_ref[...], wt_ref[pl.ds(3 * _DC, _DC), :],
                        preferred_element_type=f32))
    nl = logits + nb_ref[...]

    e = jax.lax.broadcasted_iota(jnp.int32, nl.shape, 1)
    m1 = jnp.max(nl, axis=1, keepdims=True)
    i1 = jnp.min(jnp.where(nl == m1, e, _E), axis=1, keepdims=True)
    mask1 = e == i1
    nl2 = jnp.where(mask1, -jnp.inf, nl)
    m2 = jnp.max(nl2, axis=1, keepdims=True)
    i2 = jnp.min(jnp.where(nl2 == m2, e, _E), axis=1, keepdims=True)
    mask2 = e == i2

    t = jnp.exp(m2 - m1)
    w1 = 1.0 / (1.0 + t)
    w2 = t * w1
    o_ref[...] = jnp.where(mask1, w1, jnp.where(mask2, w2, 0.0))


def kernel(x, W, b):
    n_tok, d = x.shape
    noise = jax.random.normal(jax.random.key(42), (n_tok, _E),
                              dtype=jnp.float32) * 0.1
    nb = noise + b[None, :]
    wt = W.T
    grid = (n_tok // _BT,)
    x_spec = lambda c: pl.BlockSpec((_BT, _DC), lambda i, c=c: (i, c))
    return pl.pallas_call(
        _gating_body,
        grid=grid,
        in_specs=[
            x_spec(0), x_spec(1), x_spec(2), x_spec(3),
            pl.BlockSpec((d, _E), lambda i: (0, 0)),
            pl.BlockSpec((_BT, _E), lambda i: (i, 0)),
        ],
        out_specs=pl.BlockSpec((_BT, _E), lambda i: (i, 0)),
        out_shape=jax.ShapeDtypeStruct((n_tok, _E), jnp.float32),
    )(x, x, x, x, wt, nb)
